# Initial kernel scaffold; baseline (speedup 1.0000x reference)
import functools

import jax
import jax.numpy as jnp
from jax import lax
from jax.experimental import pallas as pl
from jax.experimental.pallas import tpu as pltpu
from jax.experimental.pallas import tpu_sc as plsc

N = 10000
H = 256
E = 160000
HH = H // 2

NC = 2
NS = 16
EPT = E // NS
CH = 80
NCHUNK = EPT // CH
ZR = 125
RPT = N // NS

RB = 1000
NB = N // RB


def _mm_stats_body(nsteps, x_ref, w1_ref, b1_ref, w2_ref, b2_ref,
                   h1_ref, h2_ref, st_ref, acc_ref):
    i = pl.program_id(0)
    x = x_ref[...]
    h1 = jnp.dot(x, w1_ref[...], preferred_element_type=jnp.float32) + b1_ref[...]
    h2 = jnp.dot(x, w2_ref[...], preferred_element_type=jnp.float32) + b2_ref[...]
    h1_ref[...] = h1
    h2_ref[...] = h2
    part = jnp.concatenate([
        jnp.sum(h1, axis=0, keepdims=True),
        jnp.sum(h1 * h1, axis=0, keepdims=True),
        jnp.sum(h2, axis=0, keepdims=True),
        jnp.sum(h2 * h2, axis=0, keepdims=True),
        jnp.zeros((4, H), jnp.float32),
    ], axis=0)

    @pl.when(i == 0)
    def _():
        acc_ref[...] = part

    @pl.when(i > 0)
    def _():
        acc_ref[...] = acc_ref[...] + part

    @pl.when(i == nsteps - 1)
    def _():
        st_ref[...] = acc_ref[...]


def _mm_stats(x, w1, b1, w2, b2):
    grid = (NB,)
    return pl.pallas_call(
        functools.partial(_mm_stats_body, NB),
        grid=grid,
        in_specs=[
            pl.BlockSpec((RB, H), lambda i: (i, 0)),
            pl.BlockSpec((H, H), lambda i: (0, 0)),
            pl.BlockSpec((1, H), lambda i: (0, 0)),
            pl.BlockSpec((H, H), lambda i: (0, 0)),
            pl.BlockSpec((1, H), lambda i: (0, 0)),
        ],
        out_specs=[
            pl.BlockSpec((RB, H), lambda i: (i, 0)),
            pl.BlockSpec((RB, H), lambda i: (i, 0)),
            pl.BlockSpec((8, H), lambda i: (0, 0)),
        ],
        out_shape=[
            jax.ShapeDtypeStruct((N, H), jnp.float32),
            jax.ShapeDtypeStruct((N, H), jnp.float32),
            jax.ShapeDtypeStruct((8, H), jnp.float32),
        ],
        scratch_shapes=[pltpu.VMEM((8, H), jnp.float32)],
    )(x, w1, b1.reshape(1, H), w2, b2.reshape(1, H))


def _finish_split_body(h1_ref, h2_ref, st_ref,
                       g1_ref, be1_ref, w1b_ref, b1b_ref,
                       g2_ref, be2_ref, w2b_ref, b2b_ref,
                       y1lo_ref, y1hi_ref, y2lo_ref, y2hi_ref):
    st = st_ref[...]
    inv_n = jnp.float32(1.0 / N)

    def norm_relu(h, srow, g, be):
        mu = st[srow:srow + 1] * inv_n
        var = st[srow + 1:srow + 2] * inv_n - mu * mu
        return jax.nn.relu((h - mu) / jnp.sqrt(var + 1e-5) * g + be)

    a1 = norm_relu(h1_ref[...], 0, g1_ref[...], be1_ref[...])
    y1 = jnp.dot(a1, w1b_ref[...], preferred_element_type=jnp.float32) + b1b_ref[...]
    y1lo_ref[...] = y1[:, :HH]
    y1hi_ref[...] = y1[:, HH:]
    a2 = norm_relu(h2_ref[...], 2, g2_ref[...], be2_ref[...])
    y2 = jnp.dot(a2, w2b_ref[...], preferred_element_type=jnp.float32) + b2b_ref[...]
    y2lo_ref[...] = y2[:, :HH]
    y2hi_ref[...] = y2[:, HH:]


def _finish_split(h1, h2, st, g1, be1, w1b, b1b, g2, be2, w2b, b2b):
    grid = (NB,)
    vec = pl.BlockSpec((1, H), lambda i: (0, 0))
    mat = pl.BlockSpec((H, H), lambda i: (0, 0))
    blk = pl.BlockSpec((RB, H), lambda i: (i, 0))
    half = pl.BlockSpec((RB, HH), lambda i: (i, 0))
    return pl.pallas_call(
        _finish_split_body,
        grid=grid,
        in_specs=[blk, blk, pl.BlockSpec((8, H), lambda i: (0, 0)),
                  vec, vec, mat, vec, vec, vec, mat, vec],
        out_specs=[half, half, half, half],
        out_shape=[jax.ShapeDtypeStruct((N, HH), jnp.float32)] * 4,
    )(h1, h2, st, g1.reshape(1, H), be1.reshape(1, H), w1b,
      b1b.reshape(1, H), g2.reshape(1, H), be2.reshape(1, H), w2b,
      b2b.reshape(1, H))


def _sc_body(y1lo, y1hi, y2lo, y2hi, i0_hbm, i1_hbm, i2_hbm, out_hbm,
             idx0_v, idx1_v, idx2_v, r1, r2, prod, zbuf, accum, sem1, sem2):
    c = lax.axis_index("c")
    s = lax.axis_index("s")

    zero = jnp.zeros((16,), jnp.float32)

    def zrow(r, _):
        for j in range(HH // 16):
            zbuf[r, pl.ds(j * 16, 16)] = zero
        return 0

    lax.fori_loop(0, ZR, zrow, 0)
    for k in range(RPT // ZR):
        pltpu.sync_copy(zbuf, accum.at[pl.ds(s * RPT + k * ZR, ZR)])
    plsc.subcore_barrier()

    def process(t1, t2):
        def chunk(ch, _):
            base = s * EPT + ch * CH
            pltpu.sync_copy(i1_hbm.at[pl.ds(base, CH)], idx1_v)
            pltpu.sync_copy(i2_hbm.at[pl.ds(base, CH)], idx2_v)
            pltpu.sync_copy(i0_hbm.at[pl.ds(base, CH)], idx0_v)
            cp1 = pltpu.async_copy(t1.at[idx1_v], r1, sem1)
            cp2 = pltpu.async_copy(t2.at[idx2_v], r2, sem2)
            cp1.wait()
            cp2.wait()

            def mrow(r, _):
                for j in range(HH // 16):
                    sl = pl.ds(j * 16, 16)
                    prod[r, sl] = r1[r, sl] * r2[r, sl]
                return 0

            lax.fori_loop(0, CH, mrow, 0)
            pltpu.sync_copy(prod, accum.at[idx0_v], add=True)
            return 0

        lax.fori_loop(0, NCHUNK, chunk, 0)

    @pl.when(c == 0)
    def _():
        process(y1lo, y2lo)

    @pl.when(c == 1)
    def _():
        process(y1hi, y2hi)

    plsc.subcore_barrier()
    for k in range(RPT // ZR):
        row0 = s * RPT + k * ZR
        pltpu.sync_copy(accum.at[pl.ds(row0, ZR)],
                        out_hbm.at[c].at[pl.ds(row0, ZR)])


def _sc_edge_aggregate(y1lo, y1hi, y2lo, y2hi, i0, i1, i2):
    mesh = plsc.VectorSubcoreMesh(core_axis_name="c", subcore_axis_name="s")
    fn = pl.kernel(
        _sc_body,
        out_type=jax.ShapeDtypeStruct((NC, N, HH), jnp.float32),
        mesh=mesh,
        scratch_types=[
            pltpu.VMEM((CH,), jnp.int32),
            pltpu.VMEM((CH,), jnp.int32),
            pltpu.VMEM((CH,), jnp.int32),
            pltpu.VMEM((CH, HH), jnp.float32),
            pltpu.VMEM((CH, HH), jnp.float32),
            pltpu.VMEM((CH, HH), jnp.float32),
            pltpu.VMEM((ZR, HH), jnp.float32),
            pltpu.VMEM_SHARED((N, HH), jnp.float32),
            pltpu.SemaphoreType.DMA,
            pltpu.SemaphoreType.DMA,
        ],
    )
    return fn(y1lo, y1hi, y2lo, y2hi, i0, i1, i2)


def _upd_stats_body(nsteps, x_ref, p0_ref, p1_ref, wx_ref, w0_ref, w1_ref,
                    b_ref, hu_ref, st_ref, acc_ref):
    i = pl.program_id(0)
    hu = (jnp.dot(x_ref[...], wx_ref[...], preferred_element_type=jnp.float32)
          + jnp.dot(p0_ref[...], w0_ref[...], preferred_element_type=jnp.float32)
          + jnp.dot(p1_ref[...], w1_ref[...], preferred_element_type=jnp.float32)
          + b_ref[...])
    hu_ref[...] = hu
    part = jnp.concatenate([
        jnp.sum(hu, axis=0, keepdims=True),
        jnp.sum(hu * hu, axis=0, keepdims=True),
        jnp.zeros((6, H), jnp.float32),
    ], axis=0)

    @pl.when(i == 0)
    def _():
        acc_ref[...] = part

    @pl.when(i > 0)
    def _():
        acc_ref[...] = acc_ref[...] + part

    @pl.when(i == nsteps - 1)
    def _():
        st_ref[...] = acc_ref[...]


def _upd_stats(x, p0, p1, wx, w0, w1, b):
    grid = (NB,)
    blk = pl.BlockSpec((RB, H), lambda i: (i, 0))
    half = pl.BlockSpec((RB, HH), lambda i: (i, 0))
    return pl.pallas_call(
        functools.partial(_upd_stats_body, NB),
        grid=grid,
        in_specs=[blk, half, half,
                  pl.BlockSpec((H, H), lambda i: (0, 0)),
                  pl.BlockSpec((HH, H), lambda i: (0, 0)),
                  pl.BlockSpec((HH, H), lambda i: (0, 0)),
                  pl.BlockSpec((1, H), lambda i: (0, 0))],
        out_specs=[blk, pl.BlockSpec((8, H), lambda i: (0, 0))],
        out_shape=[
            jax.ShapeDtypeStruct((N, H), jnp.float32),
            jax.ShapeDtypeStruct((8, H), jnp.float32),
        ],
        scratch_shapes=[pltpu.VMEM((8, H), jnp.float32)],
    )(x, p0, p1, wx, w0, w1, b.reshape(1, H))


def _upd_finish_body(hu_ref, st_ref, g_ref, be_ref, w2_ref, b2_ref, x_ref,
                     out_ref):
    st = st_ref[...]
    inv_n = jnp.float32(1.0 / N)
    mu = st[0:1] * inv_n
    var = st[1:2] * inv_n - mu * mu
    a = jax.nn.relu((hu_ref[...] - mu) / jnp.sqrt(var + 1e-5) * g_ref[...]
                    + be_ref[...])
    out_ref[...] = (jnp.dot(a, w2_ref[...], preferred_element_type=jnp.float32)
                    + b2_ref[...] + x_ref[...])


def _upd_finish(hu, st, g, be, w2, b2, x):
    grid = (NB,)
    blk = pl.BlockSpec((RB, H), lambda i: (i, 0))
    vec = pl.BlockSpec((1, H), lambda i: (0, 0))
    return pl.pallas_call(
        _upd_finish_body,
        grid=grid,
        in_specs=[blk, pl.BlockSpec((8, H), lambda i: (0, 0)),
                  vec, vec, pl.BlockSpec((H, H), lambda i: (0, 0)), vec, blk],
        out_specs=blk,
        out_shape=jax.ShapeDtypeStruct((N, H), jnp.float32),
    )(hu, st, g.reshape(1, H), be.reshape(1, H), w2, b2.reshape(1, H), x)


def kernel(pair_h, tuple_index, W1a, b1a, g1, be1, W1b, b1b,
           W2a, b2a, g2, be2, W2b, b2b, Wu1, bu1, gu, beu, Wu2, bu2):
    h1, h2, st12 = _mm_stats(pair_h, W1a, b1a, W2a, b2a)
    y1lo, y1hi, y2lo, y2hi = _finish_split(
        h1, h2, st12, g1, be1, W1b, b1b, g2, be2, W2b, b2b)

    i0 = tuple_index[0]
    i1 = tuple_index[1]
    i2 = tuple_index[2]
    p = _sc_edge_aggregate(y1lo, y1hi, y2lo, y2hi, i0, i1, i2)

    hu, stu = _upd_stats(pair_h, p[0], p[1], Wu1[:H], Wu1[H:H + HH],
                         Wu1[H + HH:], bu1)
    return _upd_finish(hu, stu, gu, beu, Wu2, bu2, pair_h)


# speedup vs baseline: 3.0293x; 3.0293x over previous
import functools

import jax
import jax.numpy as jnp
from jax import lax
from jax.experimental import pallas as pl
from jax.experimental.pallas import tpu as pltpu
from jax.experimental.pallas import tpu_sc as plsc

N = 10000
H = 256
E = 160000
HH = H // 2

NC = 2
NS = 16
EPT = E // NS
CH = 80
NCHUNK = EPT // CH
NP = 10240
ZR = 128
RPT = NP // NS

RB = 1000
NB = N // RB


def _mm_stats_body(nsteps, x_ref, w1_ref, b1_ref, w2_ref, b2_ref,
                   h1_ref, h2_ref, st_ref, acc_ref):
    i = pl.program_id(0)
    x = x_ref[...]
    h1 = jnp.dot(x, w1_ref[...], preferred_element_type=jnp.float32) + b1_ref[...]
    h2 = jnp.dot(x, w2_ref[...], preferred_element_type=jnp.float32) + b2_ref[...]
    h1_ref[...] = h1
    h2_ref[...] = h2
    part = jnp.concatenate([
        jnp.sum(h1, axis=0, keepdims=True),
        jnp.sum(h1 * h1, axis=0, keepdims=True),
        jnp.sum(h2, axis=0, keepdims=True),
        jnp.sum(h2 * h2, axis=0, keepdims=True),
        jnp.zeros((4, H), jnp.float32),
    ], axis=0)

    @pl.when(i == 0)
    def _():
        acc_ref[...] = part

    @pl.when(i > 0)
    def _():
        acc_ref[...] = acc_ref[...] + part

    @pl.when(i == nsteps - 1)
    def _():
        st_ref[...] = acc_ref[...]


def _mm_stats(x, w1, b1, w2, b2):
    grid = (NB,)
    return pl.pallas_call(
        functools.partial(_mm_stats_body, NB),
        grid=grid,
        in_specs=[
            pl.BlockSpec((RB, H), lambda i: (i, 0)),
            pl.BlockSpec((H, H), lambda i: (0, 0)),
            pl.BlockSpec((1, H), lambda i: (0, 0)),
            pl.BlockSpec((H, H), lambda i: (0, 0)),
            pl.BlockSpec((1, H), lambda i: (0, 0)),
        ],
        out_specs=[
            pl.BlockSpec((RB, H), lambda i: (i, 0)),
            pl.BlockSpec((RB, H), lambda i: (i, 0)),
            pl.BlockSpec((8, H), lambda i: (0, 0)),
        ],
        out_shape=[
            jax.ShapeDtypeStruct((N, H), jnp.float32),
            jax.ShapeDtypeStruct((N, H), jnp.float32),
            jax.ShapeDtypeStruct((8, H), jnp.float32),
        ],
        scratch_shapes=[pltpu.VMEM((8, H), jnp.float32)],
    )(x, w1, b1.reshape(1, H), w2, b2.reshape(1, H))


def _finish_split_body(h1_ref, h2_ref, st_ref,
                       g1_ref, be1_ref, w1b_ref, b1b_ref,
                       g2_ref, be2_ref, w2b_ref, b2b_ref,
                       y1lo_ref, y1hi_ref, y2lo_ref, y2hi_ref):
    st = st_ref[...]
    inv_n = jnp.float32(1.0 / N)

    def norm_relu(h, srow, g, be):
        mu = st[srow:srow + 1] * inv_n
        var = st[srow + 1:srow + 2] * inv_n - mu * mu
        return jax.nn.relu((h - mu) / jnp.sqrt(var + 1e-5) * g + be)

    a1 = norm_relu(h1_ref[...], 0, g1_ref[...], be1_ref[...])
    y1 = jnp.dot(a1, w1b_ref[...], preferred_element_type=jnp.float32) + b1b_ref[...]
    y1lo_ref[...] = y1[:, :HH]
    y1hi_ref[...] = y1[:, HH:]
    a2 = norm_relu(h2_ref[...], 2, g2_ref[...], be2_ref[...])
    y2 = jnp.dot(a2, w2b_ref[...], preferred_element_type=jnp.float32) + b2b_ref[...]
    y2lo_ref[...] = y2[:, :HH]
    y2hi_ref[...] = y2[:, HH:]


def _finish_split(h1, h2, st, g1, be1, w1b, b1b, g2, be2, w2b, b2b):
    grid = (NB,)
    vec = pl.BlockSpec((1, H), lambda i: (0, 0))
    mat = pl.BlockSpec((H, H), lambda i: (0, 0))
    blk = pl.BlockSpec((RB, H), lambda i: (i, 0))
    half = pl.BlockSpec((RB, HH), lambda i: (i, 0))
    return pl.pallas_call(
        _finish_split_body,
        grid=grid,
        in_specs=[blk, blk, pl.BlockSpec((8, H), lambda i: (0, 0)),
                  vec, vec, mat, vec, vec, vec, mat, vec],
        out_specs=[half, half, half, half],
        out_shape=[jax.ShapeDtypeStruct((N, HH), jnp.float32)] * 4,
    )(h1, h2, st, g1.reshape(1, H), be1.reshape(1, H), w1b,
      b1b.reshape(1, H), g2.reshape(1, H), be2.reshape(1, H), w2b,
      b2b.reshape(1, H))


def _sc_body(y1lo, y1hi, y2lo, y2hi, i0_hbm, i1_hbm, i2_hbm, out_hbm,
             idx0_v, idx1_v, idx2_v, r1, r2, prod, zbuf, accum, sem1, sem2):
    c = lax.axis_index("c")
    s = lax.axis_index("s")

    zero = jnp.zeros((16,), jnp.float32)

    def zrow(r, _):
        for j in range(HH // 16):
            zbuf[r, pl.ds(j * 16, 16)] = zero
        return 0

    lax.fori_loop(0, ZR, zrow, 0)
    for k in range(RPT // ZR):
        pltpu.sync_copy(zbuf, accum.at[pl.ds(s * RPT + k * ZR, ZR)])
    plsc.subcore_barrier()

    def process(t1, t2):
        def chunk(ch, _):
            base = s * EPT + ch * CH
            pltpu.sync_copy(i1_hbm.at[pl.ds(base, CH)], idx1_v)
            pltpu.sync_copy(i2_hbm.at[pl.ds(base, CH)], idx2_v)
            pltpu.sync_copy(i0_hbm.at[pl.ds(base, CH)], idx0_v)
            cp1 = pltpu.async_copy(t1.at[idx1_v], r1, sem1)
            cp2 = pltpu.async_copy(t2.at[idx2_v], r2, sem2)
            cp1.wait()
            cp2.wait()

            def mrow(r, _):
                for j in range(HH // 16):
                    sl = pl.ds(j * 16, 16)
                    prod[r, sl] = r1[r, sl] * r2[r, sl]
                return 0

            lax.fori_loop(0, CH, mrow, 0)
            pltpu.sync_copy(prod, accum.at[idx0_v], add=True)
            return 0

        lax.fori_loop(0, NCHUNK, chunk, 0)

    @pl.when(c == 0)
    def _():
        process(y1lo, y2lo)

    @pl.when(c == 1)
    def _():
        process(y1hi, y2hi)

    plsc.subcore_barrier()
    for k in range(RPT // ZR):
        row0 = s * RPT + k * ZR
        pltpu.sync_copy(accum.at[pl.ds(row0, ZR)],
                        out_hbm.at[c].at[pl.ds(row0, ZR)])


def _sc_edge_aggregate(y1lo, y1hi, y2lo, y2hi, i0, i1, i2):
    mesh = plsc.VectorSubcoreMesh(core_axis_name="c", subcore_axis_name="s",
                                  num_cores=NC, num_subcores=NS)
    fn = pl.kernel(
        _sc_body,
        out_type=jax.ShapeDtypeStruct((NC, NP, HH), jnp.float32),
        mesh=mesh,
        scratch_types=[
            pltpu.VMEM((CH,), jnp.int32),
            pltpu.VMEM((CH,), jnp.int32),
            pltpu.VMEM((CH,), jnp.int32),
            pltpu.VMEM((CH, HH), jnp.float32),
            pltpu.VMEM((CH, HH), jnp.float32),
            pltpu.VMEM((CH, HH), jnp.float32),
            pltpu.VMEM((ZR, HH), jnp.float32),
            pltpu.VMEM_SHARED((NP, HH), jnp.float32),
            pltpu.SemaphoreType.DMA,
            pltpu.SemaphoreType.DMA,
        ],
    )
    return fn(y1lo, y1hi, y2lo, y2hi, i0, i1, i2)


def _upd_stats_body(nsteps, x_ref, p0_ref, p1_ref, wx_ref, w0_ref, w1_ref,
                    b_ref, hu_ref, st_ref, acc_ref):
    i = pl.program_id(0)
    hu = (jnp.dot(x_ref[...], wx_ref[...], preferred_element_type=jnp.float32)
          + jnp.dot(p0_ref[...], w0_ref[...], preferred_element_type=jnp.float32)
          + jnp.dot(p1_ref[...], w1_ref[...], preferred_element_type=jnp.float32)
          + b_ref[...])
    hu_ref[...] = hu
    part = jnp.concatenate([
        jnp.sum(hu, axis=0, keepdims=True),
        jnp.sum(hu * hu, axis=0, keepdims=True),
        jnp.zeros((6, H), jnp.float32),
    ], axis=0)

    @pl.when(i == 0)
    def _():
        acc_ref[...] = part

    @pl.when(i > 0)
    def _():
        acc_ref[...] = acc_ref[...] + part

    @pl.when(i == nsteps - 1)
    def _():
        st_ref[...] = acc_ref[...]


def _upd_stats(x, p0, p1, wx, w0, w1, b):
    grid = (NB,)
    blk = pl.BlockSpec((RB, H), lambda i: (i, 0))
    half = pl.BlockSpec((RB, HH), lambda i: (i, 0))
    return pl.pallas_call(
        functools.partial(_upd_stats_body, NB),
        grid=grid,
        in_specs=[blk, half, half,
                  pl.BlockSpec((H, H), lambda i: (0, 0)),
                  pl.BlockSpec((HH, H), lambda i: (0, 0)),
                  pl.BlockSpec((HH, H), lambda i: (0, 0)),
                  pl.BlockSpec((1, H), lambda i: (0, 0))],
        out_specs=[blk, pl.BlockSpec((8, H), lambda i: (0, 0))],
        out_shape=[
            jax.ShapeDtypeStruct((N, H), jnp.float32),
            jax.ShapeDtypeStruct((8, H), jnp.float32),
        ],
        scratch_shapes=[pltpu.VMEM((8, H), jnp.float32)],
    )(x, p0, p1, wx, w0, w1, b.reshape(1, H))


def _upd_finish_body(hu_ref, st_ref, g_ref, be_ref, w2_ref, b2_ref, x_ref,
                     out_ref):
    st = st_ref[...]
    inv_n = jnp.float32(1.0 / N)
    mu = st[0:1] * inv_n
    var = st[1:2] * inv_n - mu * mu
    a = jax.nn.relu((hu_ref[...] - mu) / jnp.sqrt(var + 1e-5) * g_ref[...]
                    + be_ref[...])
    out_ref[...] = (jnp.dot(a, w2_ref[...], preferred_element_type=jnp.float32)
                    + b2_ref[...] + x_ref[...])


def _upd_finish(hu, st, g, be, w2, b2, x):
    grid = (NB,)
    blk = pl.BlockSpec((RB, H), lambda i: (i, 0))
    vec = pl.BlockSpec((1, H), lambda i: (0, 0))
    return pl.pallas_call(
        _upd_finish_body,
        grid=grid,
        in_specs=[blk, pl.BlockSpec((8, H), lambda i: (0, 0)),
                  vec, vec, pl.BlockSpec((H, H), lambda i: (0, 0)), vec, blk],
        out_specs=blk,
        out_shape=jax.ShapeDtypeStruct((N, H), jnp.float32),
    )(hu, st, g.reshape(1, H), be.reshape(1, H), w2, b2.reshape(1, H), x)


def kernel(pair_h, tuple_index, W1a, b1a, g1, be1, W1b, b1b,
           W2a, b2a, g2, be2, W2b, b2b, Wu1, bu1, gu, beu, Wu2, bu2):
    h1, h2, st12 = _mm_stats(pair_h, W1a, b1a, W2a, b2a)
    y1lo, y1hi, y2lo, y2hi = _finish_split(
        h1, h2, st12, g1, be1, W1b, b1b, g2, be2, W2b, b2b)

    i0 = tuple_index[0]
    i1 = tuple_index[1]
    i2 = tuple_index[2]
    p = _sc_edge_aggregate(y1lo, y1hi, y2lo, y2hi, i0, i1, i2)

    hu, stu = _upd_stats(pair_h, p[0], p[1], Wu1[:H], Wu1[H:H + HH],
                         Wu1[H + HH:], bu1)
    return _upd_finish(hu, stu, gu, beu, Wu2, bu2, pair_h)


# speedup vs baseline: 5.5080x; 1.8183x over previous
import functools

import jax
import jax.numpy as jnp
from jax import lax
from jax.experimental import pallas as pl
from jax.experimental.pallas import tpu as pltpu
from jax.experimental.pallas import tpu_sc as plsc

N = 10000
H = 256
E = 160000
HH = H // 2

NC = 2
NS = 16
EPT = E // NS
CH = 40
NCHUNK = EPT // CH
SB = 50
NSB = NCHUNK // SB
NP = 10240
RPT = NP // NS

RB = 1000
NB = N // RB


def _mm_stats_body(nsteps, x_ref, w1_ref, b1_ref, w2_ref, b2_ref,
                   h1_ref, h2_ref, st_ref, acc_ref):
    i = pl.program_id(0)
    x = x_ref[...]
    h1 = jnp.dot(x, w1_ref[...], preferred_element_type=jnp.float32) + b1_ref[...]
    h2 = jnp.dot(x, w2_ref[...], preferred_element_type=jnp.float32) + b2_ref[...]
    h1_ref[...] = h1
    h2_ref[...] = h2
    part = jnp.concatenate([
        jnp.sum(h1, axis=0, keepdims=True),
        jnp.sum(h1 * h1, axis=0, keepdims=True),
        jnp.sum(h2, axis=0, keepdims=True),
        jnp.sum(h2 * h2, axis=0, keepdims=True),
        jnp.zeros((4, H), jnp.float32),
    ], axis=0)

    @pl.when(i == 0)
    def _():
        acc_ref[...] = part

    @pl.when(i > 0)
    def _():
        acc_ref[...] = acc_ref[...] + part

    @pl.when(i == nsteps - 1)
    def _():
        st_ref[...] = acc_ref[...]


def _mm_stats(x, w1, b1, w2, b2):
    grid = (NB,)
    return pl.pallas_call(
        functools.partial(_mm_stats_body, NB),
        grid=grid,
        in_specs=[
            pl.BlockSpec((RB, H), lambda i: (i, 0)),
            pl.BlockSpec((H, H), lambda i: (0, 0)),
            pl.BlockSpec((1, H), lambda i: (0, 0)),
            pl.BlockSpec((H, H), lambda i: (0, 0)),
            pl.BlockSpec((1, H), lambda i: (0, 0)),
        ],
        out_specs=[
            pl.BlockSpec((RB, H), lambda i: (i, 0)),
            pl.BlockSpec((RB, H), lambda i: (i, 0)),
            pl.BlockSpec((8, H), lambda i: (0, 0)),
        ],
        out_shape=[
            jax.ShapeDtypeStruct((N, H), jnp.float32),
            jax.ShapeDtypeStruct((N, H), jnp.float32),
            jax.ShapeDtypeStruct((8, H), jnp.float32),
        ],
        scratch_shapes=[pltpu.VMEM((8, H), jnp.float32)],
    )(x, w1, b1.reshape(1, H), w2, b2.reshape(1, H))


def _finish_split_body(h1_ref, h2_ref, st_ref,
                       g1_ref, be1_ref, w1b_ref, b1b_ref,
                       g2_ref, be2_ref, w2b_ref, b2b_ref,
                       y1lo_ref, y1hi_ref, y2lo_ref, y2hi_ref):
    st = st_ref[...]
    inv_n = jnp.float32(1.0 / N)

    def norm_relu(h, srow, g, be):
        mu = st[srow:srow + 1] * inv_n
        var = st[srow + 1:srow + 2] * inv_n - mu * mu
        return jax.nn.relu((h - mu) / jnp.sqrt(var + 1e-5) * g + be)

    a1 = norm_relu(h1_ref[...], 0, g1_ref[...], be1_ref[...])
    y1 = jnp.dot(a1, w1b_ref[...], preferred_element_type=jnp.float32) + b1b_ref[...]
    y1lo_ref[...] = y1[:, :HH]
    y1hi_ref[...] = y1[:, HH:]
    a2 = norm_relu(h2_ref[...], 2, g2_ref[...], be2_ref[...])
    y2 = jnp.dot(a2, w2b_ref[...], preferred_element_type=jnp.float32) + b2b_ref[...]
    y2lo_ref[...] = y2[:, :HH]
    y2hi_ref[...] = y2[:, HH:]


def _finish_split(h1, h2, st, g1, be1, w1b, b1b, g2, be2, w2b, b2b):
    grid = (NB,)
    vec = pl.BlockSpec((1, H), lambda i: (0, 0))
    mat = pl.BlockSpec((H, H), lambda i: (0, 0))
    blk = pl.BlockSpec((RB, H), lambda i: (i, 0))
    half = pl.BlockSpec((RB, HH), lambda i: (i, 0))
    return pl.pallas_call(
        _finish_split_body,
        grid=grid,
        in_specs=[blk, blk, pl.BlockSpec((8, H), lambda i: (0, 0)),
                  vec, vec, mat, vec, vec, vec, mat, vec],
        out_specs=[half, half, half, half],
        out_shape=[jax.ShapeDtypeStruct((N, HH), jnp.float32)] * 4,
    )(h1, h2, st, g1.reshape(1, H), be1.reshape(1, H), w1b,
      b1b.reshape(1, H), g2.reshape(1, H), be2.reshape(1, H), w2b,
      b2b.reshape(1, H))


def _sc_body(y1lo, y1hi, y2lo, y2hi, i0r, i1r, i2r, out_hbm,
             ib0, ib1, ib2, r1, r2, prod, accum, gsa, gsb, ssa, ssb):
    c = lax.axis_index("c")
    s = lax.axis_index("s")

    zero = jnp.zeros((16,), jnp.float32)

    def zrow(r, _):
        for j in range(HH // 16):
            prod[0, r, pl.ds(j * 16, 16)] = zero
        return 0

    lax.fori_loop(0, CH, zrow, 0)
    for k in range(RPT // CH):
        pltpu.sync_copy(prod.at[0], accum.at[pl.ds(s * RPT + k * CH, CH)])
    plsc.subcore_barrier()

    gsems = (gsa, gsb)
    ssems = (ssa, ssb)

    def process(t1, t2):
        def fire(ch, b):
            pltpu.async_copy(t1.at[ib1.at[pl.ds(ch * CH, CH)]], r1.at[b],
                             gsems[b])
            pltpu.async_copy(t2.at[ib2.at[pl.ds(ch * CH, CH)]], r2.at[b],
                             gsems[b])

        def wait_scatter(b):
            pltpu.make_async_copy(prod.at[b], accum.at[ib0.at[0]],
                                  ssems[b]).wait()

        def drain(ch, b):
            pltpu.make_async_copy(t1.at[ib1.at[pl.ds(ch * CH, CH)]],
                                  r1.at[b], gsems[b]).wait()
            pltpu.make_async_copy(t2.at[ib2.at[pl.ds(ch * CH, CH)]],
                                  r2.at[b], gsems[b]).wait()

            @pl.when(ch >= 2)
            def _():
                wait_scatter(b)

            @plsc.parallel_loop(0, CH, 1, unroll=2)
            def _(r):
                for j in range(HH // 16):
                    sl = pl.ds(j * 16, 16)
                    prod[b, r, sl] = r1[b, r, sl] * r2[b, r, sl]

            pltpu.async_copy(prod.at[b], accum.at[ib0.at[ch]], ssems[b],
                             add=True)

        def super_blk(sb, _):
            pltpu.sync_copy(i0r.at[s * NSB + sb], ib0)
            base = s * EPT + sb * (SB * CH)
            pltpu.sync_copy(i1r.at[pl.ds(base, SB * CH)], ib1)
            pltpu.sync_copy(i2r.at[pl.ds(base, SB * CH)], ib2)

            fire(0, 0)
            fire(1, 1)

            def pair(g, _):
                ch = 2 * g
                drain(ch, 0)

                @pl.when(ch + 2 < SB)
                def _():
                    fire(ch + 2, 0)

                drain(ch + 1, 1)

                @pl.when(ch + 3 < SB)
                def _():
                    fire(ch + 3, 1)

                return 0

            lax.fori_loop(0, SB // 2, pair, 0)
            wait_scatter(0)
            wait_scatter(1)
            return 0

        lax.fori_loop(0, NSB, super_blk, 0)

    @pl.when(c == 0)
    def _():
        process(y1lo, y2lo)

    @pl.when(c == 1)
    def _():
        process(y1hi, y2hi)

    plsc.subcore_barrier()
    for k in range(RPT // CH):
        row0 = s * RPT + k * CH
        pltpu.sync_copy(accum.at[pl.ds(row0, CH)],
                        out_hbm.at[c].at[pl.ds(row0, CH)])


def _sc_edge_aggregate(y1lo, y1hi, y2lo, y2hi, i0, i1, i2):
    mesh = plsc.VectorSubcoreMesh(core_axis_name="c", subcore_axis_name="s",
                                  num_cores=NC, num_subcores=NS)
    fn = pl.kernel(
        _sc_body,
        out_type=jax.ShapeDtypeStruct((NC, NP, HH), jnp.float32),
        mesh=mesh,
        scratch_types=[
            pltpu.VMEM((SB, CH), jnp.int32),
            pltpu.VMEM((SB * CH,), jnp.int32),
            pltpu.VMEM((SB * CH,), jnp.int32),
            pltpu.VMEM((2, CH, HH), jnp.float32),
            pltpu.VMEM((2, CH, HH), jnp.float32),
            pltpu.VMEM((2, CH, HH), jnp.float32),
            pltpu.VMEM_SHARED((NP, HH), jnp.float32),
            pltpu.SemaphoreType.DMA,
            pltpu.SemaphoreType.DMA,
            pltpu.SemaphoreType.DMA,
            pltpu.SemaphoreType.DMA,
        ],
    )
    return fn(y1lo, y1hi, y2lo, y2hi,
              i0.reshape(NS * NSB, SB, CH), i1, i2)


def _upd_stats_body(nsteps, x_ref, p0_ref, p1_ref, wx_ref, w0_ref, w1_ref,
                    b_ref, hu_ref, st_ref, acc_ref):
    i = pl.program_id(0)
    hu = (jnp.dot(x_ref[...], wx_ref[...], preferred_element_type=jnp.float32)
          + jnp.dot(p0_ref[...], w0_ref[...], preferred_element_type=jnp.float32)
          + jnp.dot(p1_ref[...], w1_ref[...], preferred_element_type=jnp.float32)
          + b_ref[...])
    hu_ref[...] = hu
    part = jnp.concatenate([
        jnp.sum(hu, axis=0, keepdims=True),
        jnp.sum(hu * hu, axis=0, keepdims=True),
        jnp.zeros((6, H), jnp.float32),
    ], axis=0)

    @pl.when(i == 0)
    def _():
        acc_ref[...] = part

    @pl.when(i > 0)
    def _():
        acc_ref[...] = acc_ref[...] + part

    @pl.when(i == nsteps - 1)
    def _():
        st_ref[...] = acc_ref[...]


def _upd_stats(x, p0, p1, wx, w0, w1, b):
    grid = (NB,)
    blk = pl.BlockSpec((RB, H), lambda i: (i, 0))
    half = pl.BlockSpec((RB, HH), lambda i: (i, 0))
    return pl.pallas_call(
        functools.partial(_upd_stats_body, NB),
        grid=grid,
        in_specs=[blk, half, half,
                  pl.BlockSpec((H, H), lambda i: (0, 0)),
                  pl.BlockSpec((HH, H), lambda i: (0, 0)),
                  pl.BlockSpec((HH, H), lambda i: (0, 0)),
                  pl.BlockSpec((1, H), lambda i: (0, 0))],
        out_specs=[blk, pl.BlockSpec((8, H), lambda i: (0, 0))],
        out_shape=[
            jax.ShapeDtypeStruct((N, H), jnp.float32),
            jax.ShapeDtypeStruct((8, H), jnp.float32),
        ],
        scratch_shapes=[pltpu.VMEM((8, H), jnp.float32)],
    )(x, p0, p1, wx, w0, w1, b.reshape(1, H))


def _upd_finish_body(hu_ref, st_ref, g_ref, be_ref, w2_ref, b2_ref, x_ref,
                     out_ref):
    st = st_ref[...]
    inv_n = jnp.float32(1.0 / N)
    mu = st[0:1] * inv_n
    var = st[1:2] * inv_n - mu * mu
    a = jax.nn.relu((hu_ref[...] - mu) / jnp.sqrt(var + 1e-5) * g_ref[...]
                    + be_ref[...])
    out_ref[...] = (jnp.dot(a, w2_ref[...], preferred_element_type=jnp.float32)
                    + b2_ref[...] + x_ref[...])


def _upd_finish(hu, st, g, be, w2, b2, x):
    grid = (NB,)
    blk = pl.BlockSpec((RB, H), lambda i: (i, 0))
    vec = pl.BlockSpec((1, H), lambda i: (0, 0))
    return pl.pallas_call(
        _upd_finish_body,
        grid=grid,
        in_specs=[blk, pl.BlockSpec((8, H), lambda i: (0, 0)),
                  vec, vec, pl.BlockSpec((H, H), lambda i: (0, 0)), vec, blk],
        out_specs=blk,
        out_shape=jax.ShapeDtypeStruct((N, H), jnp.float32),
    )(hu, st, g.reshape(1, H), be.reshape(1, H), w2, b2.reshape(1, H), x)


def kernel(pair_h, tuple_index, W1a, b1a, g1, be1, W1b, b1b,
           W2a, b2a, g2, be2, W2b, b2b, Wu1, bu1, gu, beu, Wu2, bu2):
    h1, h2, st12 = _mm_stats(pair_h, W1a, b1a, W2a, b2a)
    y1lo, y1hi, y2lo, y2hi = _finish_split(
        h1, h2, st12, g1, be1, W1b, b1b, g2, be2, W2b, b2b)

    i0 = tuple_index[0]
    i1 = tuple_index[1]
    i2 = tuple_index[2]
    p = _sc_edge_aggregate(y1lo, y1hi, y2lo, y2hi, i0, i1, i2)

    hu, stu = _upd_stats(pair_h, p[0], p[1], Wu1[:H], Wu1[H:H + HH],
                         Wu1[H + HH:], bu1)
    return _upd_finish(hu, stu, gu, beu, Wu2, bu2, pair_h)


# speedup vs baseline: 5.6507x; 1.0259x over previous
import functools

import jax
import jax.numpy as jnp
from jax import lax
from jax.experimental import pallas as pl
from jax.experimental.pallas import tpu as pltpu
from jax.experimental.pallas import tpu_sc as plsc

N = 10000
H = 256
E = 160000
HH = H // 2

NC = 2
NS = 16
EPT = E // NS
CH = 40
NCHUNK = EPT // CH
SB = 50
NSB = NCHUNK // SB
NP = 10240
RPT = NP // NS

RB = 2000
NB = N // RB


def _mm_stats_body(nsteps, x_ref, w1_ref, b1_ref, w2_ref, b2_ref,
                   h1_ref, h2_ref, st_ref, acc_ref):
    i = pl.program_id(0)
    x = x_ref[...]
    h1 = jnp.dot(x, w1_ref[...], preferred_element_type=jnp.float32) + b1_ref[...]
    h2 = jnp.dot(x, w2_ref[...], preferred_element_type=jnp.float32) + b2_ref[...]
    h1_ref[...] = h1
    h2_ref[...] = h2
    part = jnp.concatenate([
        jnp.sum(h1, axis=0, keepdims=True),
        jnp.sum(h1 * h1, axis=0, keepdims=True),
        jnp.sum(h2, axis=0, keepdims=True),
        jnp.sum(h2 * h2, axis=0, keepdims=True),
        jnp.zeros((4, H), jnp.float32),
    ], axis=0)

    @pl.when(i == 0)
    def _():
        acc_ref[...] = part

    @pl.when(i > 0)
    def _():
        acc_ref[...] = acc_ref[...] + part

    @pl.when(i == nsteps - 1)
    def _():
        st_ref[...] = acc_ref[...]


def _mm_stats(x, w1, b1, w2, b2):
    grid = (NB,)
    return pl.pallas_call(
        functools.partial(_mm_stats_body, NB),
        grid=grid,
        in_specs=[
            pl.BlockSpec((RB, H), lambda i: (i, 0)),
            pl.BlockSpec((H, H), lambda i: (0, 0)),
            pl.BlockSpec((1, H), lambda i: (0, 0)),
            pl.BlockSpec((H, H), lambda i: (0, 0)),
            pl.BlockSpec((1, H), lambda i: (0, 0)),
        ],
        out_specs=[
            pl.BlockSpec((RB, H), lambda i: (i, 0)),
            pl.BlockSpec((RB, H), lambda i: (i, 0)),
            pl.BlockSpec((8, H), lambda i: (0, 0)),
        ],
        out_shape=[
            jax.ShapeDtypeStruct((N, H), jnp.float32),
            jax.ShapeDtypeStruct((N, H), jnp.float32),
            jax.ShapeDtypeStruct((8, H), jnp.float32),
        ],
        scratch_shapes=[pltpu.VMEM((8, H), jnp.float32)],
    )(x, w1, b1.reshape(1, H), w2, b2.reshape(1, H))


def _finish_split_body(h1_ref, h2_ref, st_ref,
                       g1_ref, be1_ref, w1b_ref, b1b_ref,
                       g2_ref, be2_ref, w2b_ref, b2b_ref,
                       y1lo_ref, y1hi_ref, y2lo_ref, y2hi_ref):
    st = st_ref[...]
    inv_n = jnp.float32(1.0 / N)

    def norm_relu(h, srow, g, be):
        mu = st[srow:srow + 1] * inv_n
        var = st[srow + 1:srow + 2] * inv_n - mu * mu
        return jax.nn.relu((h - mu) / jnp.sqrt(var + 1e-5) * g + be)

    a1 = norm_relu(h1_ref[...], 0, g1_ref[...], be1_ref[...])
    y1 = jnp.dot(a1, w1b_ref[...], preferred_element_type=jnp.float32) + b1b_ref[...]
    y1lo_ref[...] = y1[:, :HH]
    y1hi_ref[...] = y1[:, HH:]
    a2 = norm_relu(h2_ref[...], 2, g2_ref[...], be2_ref[...])
    y2 = jnp.dot(a2, w2b_ref[...], preferred_element_type=jnp.float32) + b2b_ref[...]
    y2lo_ref[...] = y2[:, :HH]
    y2hi_ref[...] = y2[:, HH:]


def _finish_split(h1, h2, st, g1, be1, w1b, b1b, g2, be2, w2b, b2b):
    grid = (NB,)
    vec = pl.BlockSpec((1, H), lambda i: (0, 0))
    mat = pl.BlockSpec((H, H), lambda i: (0, 0))
    blk = pl.BlockSpec((RB, H), lambda i: (i, 0))
    half = pl.BlockSpec((RB, HH), lambda i: (i, 0))
    return pl.pallas_call(
        _finish_split_body,
        grid=grid,
        in_specs=[blk, blk, pl.BlockSpec((8, H), lambda i: (0, 0)),
                  vec, vec, mat, vec, vec, vec, mat, vec],
        out_specs=[half, half, half, half],
        out_shape=[jax.ShapeDtypeStruct((N, HH), jnp.float32)] * 4,
    )(h1, h2, st, g1.reshape(1, H), be1.reshape(1, H), w1b,
      b1b.reshape(1, H), g2.reshape(1, H), be2.reshape(1, H), w2b,
      b2b.reshape(1, H))


def _sc_body(y1lo, y1hi, y2lo, y2hi, i0r, i1r, i2r, out_hbm,
             ib0, ib1, ib2, r1, r2, prod, accum, gsa, gsb, ssa, ssb):
    c = lax.axis_index("c")
    s = lax.axis_index("s")

    zero = jnp.zeros((16,), jnp.float32)

    def zrow(r, _):
        for j in range(HH // 16):
            prod[0, r, pl.ds(j * 16, 16)] = zero
        return 0

    lax.fori_loop(0, CH, zrow, 0)
    for k in range(RPT // CH):
        pltpu.sync_copy(prod.at[0], accum.at[pl.ds(s * RPT + k * CH, CH)])
    plsc.subcore_barrier()

    gsems = (gsa, gsb)
    ssems = (ssa, ssb)

    def process(t1, t2):
        def fire(ch, b):
            pltpu.async_copy(t1.at[ib1.at[pl.ds(ch * CH, CH)]], r1.at[b],
                             gsems[b])
            pltpu.async_copy(t2.at[ib2.at[pl.ds(ch * CH, CH)]], r2.at[b],
                             gsems[b])

        def wait_scatter(b):
            pltpu.make_async_copy(prod.at[b], accum.at[ib0.at[0]],
                                  ssems[b]).wait()

        def drain(ch, b):
            pltpu.make_async_copy(t1.at[ib1.at[pl.ds(ch * CH, CH)]],
                                  r1.at[b], gsems[b]).wait()
            pltpu.make_async_copy(t2.at[ib2.at[pl.ds(ch * CH, CH)]],
                                  r2.at[b], gsems[b]).wait()

            @pl.when(ch >= 2)
            def _():
                wait_scatter(b)

            @plsc.parallel_loop(0, CH, 1, unroll=4)
            def _(r):
                for j in range(HH // 16):
                    sl = pl.ds(j * 16, 16)
                    prod[b, r, sl] = r1[b, r, sl] * r2[b, r, sl]

            pltpu.async_copy(prod.at[b], accum.at[ib0.at[ch]], ssems[b],
                             add=True)

        def super_blk(sb, _):
            pltpu.sync_copy(i0r.at[s * NSB + sb], ib0)
            base = s * EPT + sb * (SB * CH)
            pltpu.sync_copy(i1r.at[pl.ds(base, SB * CH)], ib1)
            pltpu.sync_copy(i2r.at[pl.ds(base, SB * CH)], ib2)

            fire(0, 0)
            fire(1, 1)

            def pair(g, _):
                ch = 2 * g
                drain(ch, 0)

                @pl.when(ch + 2 < SB)
                def _():
                    fire(ch + 2, 0)

                drain(ch + 1, 1)

                @pl.when(ch + 3 < SB)
                def _():
                    fire(ch + 3, 1)

                return 0

            lax.fori_loop(0, SB // 2, pair, 0)
            wait_scatter(0)
            wait_scatter(1)
            return 0

        lax.fori_loop(0, NSB, super_blk, 0)

    @pl.when(c == 0)
    def _():
        process(y1lo, y2lo)

    @pl.when(c == 1)
    def _():
        process(y1hi, y2hi)

    plsc.subcore_barrier()
    for k in range(RPT // CH):
        row0 = s * RPT + k * CH
        pltpu.sync_copy(accum.at[pl.ds(row0, CH)],
                        out_hbm.at[c].at[pl.ds(row0, CH)])


def _sc_edge_aggregate(y1lo, y1hi, y2lo, y2hi, i0, i1, i2):
    mesh = plsc.VectorSubcoreMesh(core_axis_name="c", subcore_axis_name="s",
                                  num_cores=NC, num_subcores=NS)
    fn = pl.kernel(
        _sc_body,
        out_type=jax.ShapeDtypeStruct((NC, NP, HH), jnp.float32),
        mesh=mesh,
        scratch_types=[
            pltpu.VMEM((SB, CH), jnp.int32),
            pltpu.VMEM((SB * CH,), jnp.int32),
            pltpu.VMEM((SB * CH,), jnp.int32),
            pltpu.VMEM((2, CH, HH), jnp.float32),
            pltpu.VMEM((2, CH, HH), jnp.float32),
            pltpu.VMEM((2, CH, HH), jnp.float32),
            pltpu.VMEM_SHARED((NP, HH), jnp.float32),
            pltpu.SemaphoreType.DMA,
            pltpu.SemaphoreType.DMA,
            pltpu.SemaphoreType.DMA,
            pltpu.SemaphoreType.DMA,
        ],
    )
    return fn(y1lo, y1hi, y2lo, y2hi,
              i0.reshape(NS * NSB, SB, CH), i1, i2)


def _upd_stats_body(nsteps, x_ref, p0_ref, p1_ref, wx_ref, w0_ref, w1_ref,
                    b_ref, hu_ref, st_ref, acc_ref):
    i = pl.program_id(0)
    hu = (jnp.dot(x_ref[...], wx_ref[...], preferred_element_type=jnp.float32)
          + jnp.dot(p0_ref[...], w0_ref[...], preferred_element_type=jnp.float32)
          + jnp.dot(p1_ref[...], w1_ref[...], preferred_element_type=jnp.float32)
          + b_ref[...])
    hu_ref[...] = hu
    part = jnp.concatenate([
        jnp.sum(hu, axis=0, keepdims=True),
        jnp.sum(hu * hu, axis=0, keepdims=True),
        jnp.zeros((6, H), jnp.float32),
    ], axis=0)

    @pl.when(i == 0)
    def _():
        acc_ref[...] = part

    @pl.when(i > 0)
    def _():
        acc_ref[...] = acc_ref[...] + part

    @pl.when(i == nsteps - 1)
    def _():
        st_ref[...] = acc_ref[...]


def _upd_stats(x, p0, p1, wx, w0, w1, b):
    grid = (NB,)
    blk = pl.BlockSpec((RB, H), lambda i: (i, 0))
    half = pl.BlockSpec((RB, HH), lambda i: (i, 0))
    return pl.pallas_call(
        functools.partial(_upd_stats_body, NB),
        grid=grid,
        in_specs=[blk, half, half,
                  pl.BlockSpec((H, H), lambda i: (0, 0)),
                  pl.BlockSpec((HH, H), lambda i: (0, 0)),
                  pl.BlockSpec((HH, H), lambda i: (0, 0)),
                  pl.BlockSpec((1, H), lambda i: (0, 0))],
        out_specs=[blk, pl.BlockSpec((8, H), lambda i: (0, 0))],
        out_shape=[
            jax.ShapeDtypeStruct((N, H), jnp.float32),
            jax.ShapeDtypeStruct((8, H), jnp.float32),
        ],
        scratch_shapes=[pltpu.VMEM((8, H), jnp.float32)],
    )(x, p0, p1, wx, w0, w1, b.reshape(1, H))


def _upd_finish_body(hu_ref, st_ref, g_ref, be_ref, w2_ref, b2_ref, x_ref,
                     out_ref):
    st = st_ref[...]
    inv_n = jnp.float32(1.0 / N)
    mu = st[0:1] * inv_n
    var = st[1:2] * inv_n - mu * mu
    a = jax.nn.relu((hu_ref[...] - mu) / jnp.sqrt(var + 1e-5) * g_ref[...]
                    + be_ref[...])
    out_ref[...] = (jnp.dot(a, w2_ref[...], preferred_element_type=jnp.float32)
                    + b2_ref[...] + x_ref[...])


def _upd_finish(hu, st, g, be, w2, b2, x):
    grid = (NB,)
    blk = pl.BlockSpec((RB, H), lambda i: (i, 0))
    vec = pl.BlockSpec((1, H), lambda i: (0, 0))
    return pl.pallas_call(
        _upd_finish_body,
        grid=grid,
        in_specs=[blk, pl.BlockSpec((8, H), lambda i: (0, 0)),
                  vec, vec, pl.BlockSpec((H, H), lambda i: (0, 0)), vec, blk],
        out_specs=blk,
        out_shape=jax.ShapeDtypeStruct((N, H), jnp.float32),
    )(hu, st, g.reshape(1, H), be.reshape(1, H), w2, b2.reshape(1, H), x)


def kernel(pair_h, tuple_index, W1a, b1a, g1, be1, W1b, b1b,
           W2a, b2a, g2, be2, W2b, b2b, Wu1, bu1, gu, beu, Wu2, bu2):
    h1, h2, st12 = _mm_stats(pair_h, W1a, b1a, W2a, b2a)
    y1lo, y1hi, y2lo, y2hi = _finish_split(
        h1, h2, st12, g1, be1, W1b, b1b, g2, be2, W2b, b2b)

    i0 = tuple_index[0]
    i1 = tuple_index[1]
    i2 = tuple_index[2]
    p = _sc_edge_aggregate(y1lo, y1hi, y2lo, y2hi, i0, i1, i2)

    hu, stu = _upd_stats(pair_h, p[0], p[1], Wu1[:H], Wu1[H:H + HH],
                         Wu1[H + HH:], bu1)
    return _upd_finish(hu, stu, gu, beu, Wu2, bu2, pair_h)


# speedup vs baseline: 6.5920x; 1.1666x over previous
import functools

import jax
import jax.numpy as jnp
from jax import lax
from jax.experimental import pallas as pl
from jax.experimental.pallas import tpu as pltpu
from jax.experimental.pallas import tpu_sc as plsc

N = 10000
H = 256
E = 160000
HH = H // 2
HQ = H // 4

NC = 2
NS = 16
EPT = E // NS
CH = 40
NCHUNK = EPT // CH
SB = 50
NSB = NCHUNK // SB
NP = 10240
RPT = NP // NS

RB = 2000
NB = N // RB


def _mm_stats_body(nsteps, x_ref, w1_ref, b1_ref, w2_ref, b2_ref,
                   h1_ref, h2_ref, st_ref, acc_ref):
    i = pl.program_id(0)
    x = x_ref[...]
    h1 = jnp.dot(x, w1_ref[...], preferred_element_type=jnp.float32) + b1_ref[...]
    h2 = jnp.dot(x, w2_ref[...], preferred_element_type=jnp.float32) + b2_ref[...]
    h1_ref[...] = h1
    h2_ref[...] = h2
    part = jnp.concatenate([
        jnp.sum(h1, axis=0, keepdims=True),
        jnp.sum(h1 * h1, axis=0, keepdims=True),
        jnp.sum(h2, axis=0, keepdims=True),
        jnp.sum(h2 * h2, axis=0, keepdims=True),
        jnp.zeros((4, H), jnp.float32),
    ], axis=0)

    @pl.when(i == 0)
    def _():
        acc_ref[...] = part

    @pl.when(i > 0)
    def _():
        acc_ref[...] = acc_ref[...] + part

    @pl.when(i == nsteps - 1)
    def _():
        st_ref[...] = acc_ref[...]


def _mm_stats(x, w1, b1, w2, b2):
    grid = (NB,)
    return pl.pallas_call(
        functools.partial(_mm_stats_body, NB),
        grid=grid,
        in_specs=[
            pl.BlockSpec((RB, H), lambda i: (i, 0)),
            pl.BlockSpec((H, H), lambda i: (0, 0)),
            pl.BlockSpec((1, H), lambda i: (0, 0)),
            pl.BlockSpec((H, H), lambda i: (0, 0)),
            pl.BlockSpec((1, H), lambda i: (0, 0)),
        ],
        out_specs=[
            pl.BlockSpec((RB, H), lambda i: (i, 0)),
            pl.BlockSpec((RB, H), lambda i: (i, 0)),
            pl.BlockSpec((8, H), lambda i: (0, 0)),
        ],
        out_shape=[
            jax.ShapeDtypeStruct((N, H), jnp.float32),
            jax.ShapeDtypeStruct((N, H), jnp.float32),
            jax.ShapeDtypeStruct((8, H), jnp.float32),
        ],
        scratch_shapes=[pltpu.VMEM((8, H), jnp.float32)],
    )(x, w1, b1.reshape(1, H), w2, b2.reshape(1, H))


def _pack_half(yh):
    a = lax.bitcast_convert_type(yh[:, :HQ].astype(jnp.bfloat16),
                                 jnp.uint16).astype(jnp.uint32)
    b = lax.bitcast_convert_type(yh[:, HQ:].astype(jnp.bfloat16),
                                 jnp.uint16).astype(jnp.uint32)
    return a | (b << 16)


def _finish_split_body(h1_ref, h2_ref, st_ref,
                       g1_ref, be1_ref, w1b_ref, b1b_ref,
                       g2_ref, be2_ref, w2b_ref, b2b_ref,
                       y1lo_ref, y1hi_ref, y2lo_ref, y2hi_ref):
    st = st_ref[...]
    inv_n = jnp.float32(1.0 / N)

    def norm_relu(h, srow, g, be):
        mu = st[srow:srow + 1] * inv_n
        var = st[srow + 1:srow + 2] * inv_n - mu * mu
        return jax.nn.relu((h - mu) / jnp.sqrt(var + 1e-5) * g + be)

    a1 = norm_relu(h1_ref[...], 0, g1_ref[...], be1_ref[...])
    y1 = jnp.dot(a1, w1b_ref[...], preferred_element_type=jnp.float32) + b1b_ref[...]
    y1lo_ref[...] = _pack_half(y1[:, :HH])
    y1hi_ref[...] = _pack_half(y1[:, HH:])
    a2 = norm_relu(h2_ref[...], 2, g2_ref[...], be2_ref[...])
    y2 = jnp.dot(a2, w2b_ref[...], preferred_element_type=jnp.float32) + b2b_ref[...]
    y2lo_ref[...] = _pack_half(y2[:, :HH])
    y2hi_ref[...] = _pack_half(y2[:, HH:])


def _finish_split(h1, h2, st, g1, be1, w1b, b1b, g2, be2, w2b, b2b):
    grid = (NB,)
    vec = pl.BlockSpec((1, H), lambda i: (0, 0))
    mat = pl.BlockSpec((H, H), lambda i: (0, 0))
    blk = pl.BlockSpec((RB, H), lambda i: (i, 0))
    half = pl.BlockSpec((RB, HQ), lambda i: (i, 0))
    return pl.pallas_call(
        _finish_split_body,
        grid=grid,
        in_specs=[blk, blk, pl.BlockSpec((8, H), lambda i: (0, 0)),
                  vec, vec, mat, vec, vec, vec, mat, vec],
        out_specs=[half, half, half, half],
        out_shape=[jax.ShapeDtypeStruct((N, HQ), jnp.uint32)] * 4,
    )(h1, h2, st, g1.reshape(1, H), be1.reshape(1, H), w1b,
      b1b.reshape(1, H), g2.reshape(1, H), be2.reshape(1, H), w2b,
      b2b.reshape(1, H))


def _sc_body(y1lo, y1hi, y2lo, y2hi, i0r, i1r, i2r, out_hbm,
             ib0, ib1, ib2, r1, r2, prod, accum, gsa, gsb, ssa, ssb):
    c = lax.axis_index("c")
    s = lax.axis_index("s")

    zero = jnp.zeros((16,), jnp.float32)

    def zrow(r, _):
        for j in range(HH // 16):
            prod[0, r, pl.ds(j * 16, 16)] = zero
        return 0

    lax.fori_loop(0, CH, zrow, 0)
    for k in range(RPT // CH):
        pltpu.sync_copy(prod.at[0], accum.at[pl.ds(s * RPT + k * CH, CH)])
    plsc.subcore_barrier()

    gsems = (gsa, gsb)
    ssems = (ssa, ssb)

    def process(t1, t2):
        def fire(ch, b):
            pltpu.async_copy(t1.at[ib1.at[pl.ds(ch * CH, CH)]], r1.at[b],
                             gsems[b])
            pltpu.async_copy(t2.at[ib2.at[pl.ds(ch * CH, CH)]], r2.at[b],
                             gsems[b])

        def wait_scatter(b):
            pltpu.make_async_copy(prod.at[b], accum.at[ib0.at[0]],
                                  ssems[b]).wait()

        def drain(ch, b):
            pltpu.make_async_copy(t1.at[ib1.at[pl.ds(ch * CH, CH)]],
                                  r1.at[b], gsems[b]).wait()
            pltpu.make_async_copy(t2.at[ib2.at[pl.ds(ch * CH, CH)]],
                                  r2.at[b], gsems[b]).wait()

            @pl.when(ch >= 2)
            def _():
                wait_scatter(b)

            @plsc.parallel_loop(0, CH, 1, unroll=4)
            def _(r):
                m = jnp.uint32(0xFFFF0000)
                for j in range(HQ // 16):
                    sl = pl.ds(j * 16, 16)
                    v1 = r1[b, r, sl]
                    v2 = r2[b, r, sl]
                    a1 = lax.bitcast_convert_type(v1 << 16, jnp.float32)
                    a2 = lax.bitcast_convert_type(v2 << 16, jnp.float32)
                    b1 = lax.bitcast_convert_type(v1 & m, jnp.float32)
                    b2 = lax.bitcast_convert_type(v2 & m, jnp.float32)
                    prod[b, r, pl.ds(j * 16, 16)] = a1 * a2
                    prod[b, r, pl.ds(HQ + j * 16, 16)] = b1 * b2

            pltpu.async_copy(prod.at[b], accum.at[ib0.at[ch]], ssems[b],
                             add=True)

        def super_blk(sb, _):
            pltpu.sync_copy(i0r.at[s * NSB + sb], ib0)
            base = s * EPT + sb * (SB * CH)
            pltpu.sync_copy(i1r.at[pl.ds(base, SB * CH)], ib1)
            pltpu.sync_copy(i2r.at[pl.ds(base, SB * CH)], ib2)

            fire(0, 0)
            fire(1, 1)

            def pair(g, _):
                ch = 2 * g
                drain(ch, 0)

                @pl.when(ch + 2 < SB)
                def _():
                    fire(ch + 2, 0)

                drain(ch + 1, 1)

                @pl.when(ch + 3 < SB)
                def _():
                    fire(ch + 3, 1)

                return 0

            lax.fori_loop(0, SB // 2, pair, 0)
            wait_scatter(0)
            wait_scatter(1)
            return 0

        lax.fori_loop(0, NSB, super_blk, 0)

    @pl.when(c == 0)
    def _():
        process(y1lo, y2lo)

    @pl.when(c == 1)
    def _():
        process(y1hi, y2hi)

    plsc.subcore_barrier()
    for k in range(RPT // CH):
        row0 = s * RPT + k * CH
        pltpu.sync_copy(accum.at[pl.ds(row0, CH)],
                        out_hbm.at[c].at[pl.ds(row0, CH)])


def _sc_edge_aggregate(y1lo, y1hi, y2lo, y2hi, i0, i1, i2):
    mesh = plsc.VectorSubcoreMesh(core_axis_name="c", subcore_axis_name="s",
                                  num_cores=NC, num_subcores=NS)
    fn = pl.kernel(
        _sc_body,
        out_type=jax.ShapeDtypeStruct((NC, NP, HH), jnp.float32),
        mesh=mesh,
        compiler_params=pltpu.CompilerParams(use_tc_tiling_on_sc=False),
        scratch_types=[
            pltpu.VMEM((SB, CH), jnp.int32),
            pltpu.VMEM((SB * CH,), jnp.int32),
            pltpu.VMEM((SB * CH,), jnp.int32),
            pltpu.VMEM((2, CH, HQ), jnp.uint32),
            pltpu.VMEM((2, CH, HQ), jnp.uint32),
            pltpu.VMEM((2, CH, HH), jnp.float32),
            pltpu.VMEM_SHARED((NP, HH), jnp.float32),
            pltpu.SemaphoreType.DMA,
            pltpu.SemaphoreType.DMA,
            pltpu.SemaphoreType.DMA,
            pltpu.SemaphoreType.DMA,
        ],
    )
    return fn(y1lo, y1hi, y2lo, y2hi,
              i0.reshape(NS * NSB, SB, CH), i1, i2)


def _upd_stats_body(nsteps, x_ref, p0_ref, p1_ref, wx_ref, w0_ref, w1_ref,
                    b_ref, hu_ref, st_ref, acc_ref):
    i = pl.program_id(0)
    hu = (jnp.dot(x_ref[...], wx_ref[...], preferred_element_type=jnp.float32)
          + jnp.dot(p0_ref[...], w0_ref[...], preferred_element_type=jnp.float32)
          + jnp.dot(p1_ref[...], w1_ref[...], preferred_element_type=jnp.float32)
          + b_ref[...])
    hu_ref[...] = hu
    part = jnp.concatenate([
        jnp.sum(hu, axis=0, keepdims=True),
        jnp.sum(hu * hu, axis=0, keepdims=True),
        jnp.zeros((6, H), jnp.float32),
    ], axis=0)

    @pl.when(i == 0)
    def _():
        acc_ref[...] = part

    @pl.when(i > 0)
    def _():
        acc_ref[...] = acc_ref[...] + part

    @pl.when(i == nsteps - 1)
    def _():
        st_ref[...] = acc_ref[...]


def _upd_stats(x, p0, p1, wx, w0, w1, b):
    grid = (NB,)
    blk = pl.BlockSpec((RB, H), lambda i: (i, 0))
    half = pl.BlockSpec((RB, HH), lambda i: (i, 0))
    return pl.pallas_call(
        functools.partial(_upd_stats_body, NB),
        grid=grid,
        in_specs=[blk, half, half,
                  pl.BlockSpec((H, H), lambda i: (0, 0)),
                  pl.BlockSpec((HH, H), lambda i: (0, 0)),
                  pl.BlockSpec((HH, H), lambda i: (0, 0)),
                  pl.BlockSpec((1, H), lambda i: (0, 0))],
        out_specs=[blk, pl.BlockSpec((8, H), lambda i: (0, 0))],
        out_shape=[
            jax.ShapeDtypeStruct((N, H), jnp.float32),
            jax.ShapeDtypeStruct((8, H), jnp.float32),
        ],
        scratch_shapes=[pltpu.VMEM((8, H), jnp.float32)],
    )(x, p0, p1, wx, w0, w1, b.reshape(1, H))


def _upd_finish_body(hu_ref, st_ref, g_ref, be_ref, w2_ref, b2_ref, x_ref,
                     out_ref):
    st = st_ref[...]
    inv_n = jnp.float32(1.0 / N)
    mu = st[0:1] * inv_n
    var = st[1:2] * inv_n - mu * mu
    a = jax.nn.relu((hu_ref[...] - mu) / jnp.sqrt(var + 1e-5) * g_ref[...]
                    + be_ref[...])
    out_ref[...] = (jnp.dot(a, w2_ref[...], preferred_element_type=jnp.float32)
                    + b2_ref[...] + x_ref[...])


def _upd_finish(hu, st, g, be, w2, b2, x):
    grid = (NB,)
    blk = pl.BlockSpec((RB, H), lambda i: (i, 0))
    vec = pl.BlockSpec((1, H), lambda i: (0, 0))
    return pl.pallas_call(
        _upd_finish_body,
        grid=grid,
        in_specs=[blk, pl.BlockSpec((8, H), lambda i: (0, 0)),
                  vec, vec, pl.BlockSpec((H, H), lambda i: (0, 0)), vec, blk],
        out_specs=blk,
        out_shape=jax.ShapeDtypeStruct((N, H), jnp.float32),
    )(hu, st, g.reshape(1, H), be.reshape(1, H), w2, b2.reshape(1, H), x)


def kernel(pair_h, tuple_index, W1a, b1a, g1, be1, W1b, b1b,
           W2a, b2a, g2, be2, W2b, b2b, Wu1, bu1, gu, beu, Wu2, bu2):
    h1, h2, st12 = _mm_stats(pair_h, W1a, b1a, W2a, b2a)
    y1lo, y1hi, y2lo, y2hi = _finish_split(
        h1, h2, st12, g1, be1, W1b, b1b, g2, be2, W2b, b2b)

    i0 = tuple_index[0]
    i1 = tuple_index[1]
    i2 = tuple_index[2]
    p = _sc_edge_aggregate(y1lo, y1hi, y2lo, y2hi, i0, i1, i2)

    hu, stu = _upd_stats(pair_h, p[0], p[1], Wu1[:H], Wu1[H:H + HH],
                         Wu1[H + HH:], bu1)
    return _upd_finish(hu, stu, gu, beu, Wu2, bu2, pair_h)


# speedup vs baseline: 7.0624x; 1.0714x over previous
import functools

import jax
import jax.numpy as jnp
from jax import lax
from jax.experimental import pallas as pl
from jax.experimental.pallas import tpu as pltpu
from jax.experimental.pallas import tpu_sc as plsc

N = 10000
H = 256
E = 160000
HH = H // 2
HQ = H // 4

NC = 2
NS = 16
EPT = E // NS
CH = 80
NCHUNK = EPT // CH
SB = 25
NSB = NCHUNK // SB
NP = 10240
RPT = NP // NS

RB = 2000
NB = N // RB


def _mm_stats_body(nsteps, x_ref, w1_ref, b1_ref, w2_ref, b2_ref,
                   h1_ref, h2_ref, st_ref, acc_ref):
    i = pl.program_id(0)
    x = x_ref[...]
    h1 = jnp.dot(x, w1_ref[...], preferred_element_type=jnp.float32) + b1_ref[...]
    h2 = jnp.dot(x, w2_ref[...], preferred_element_type=jnp.float32) + b2_ref[...]
    h1_ref[...] = h1
    h2_ref[...] = h2
    part = jnp.concatenate([
        jnp.sum(h1, axis=0, keepdims=True),
        jnp.sum(h1 * h1, axis=0, keepdims=True),
        jnp.sum(h2, axis=0, keepdims=True),
        jnp.sum(h2 * h2, axis=0, keepdims=True),
        jnp.zeros((4, H), jnp.float32),
    ], axis=0)

    @pl.when(i == 0)
    def _():
        acc_ref[...] = part

    @pl.when(i > 0)
    def _():
        acc_ref[...] = acc_ref[...] + part

    @pl.when(i == nsteps - 1)
    def _():
        st_ref[...] = acc_ref[...]


def _mm_stats(x, w1, b1, w2, b2):
    grid = (NB,)
    return pl.pallas_call(
        functools.partial(_mm_stats_body, NB),
        grid=grid,
        in_specs=[
            pl.BlockSpec((RB, H), lambda i: (i, 0)),
            pl.BlockSpec((H, H), lambda i: (0, 0)),
            pl.BlockSpec((1, H), lambda i: (0, 0)),
            pl.BlockSpec((H, H), lambda i: (0, 0)),
            pl.BlockSpec((1, H), lambda i: (0, 0)),
        ],
        out_specs=[
            pl.BlockSpec((RB, H), lambda i: (i, 0)),
            pl.BlockSpec((RB, H), lambda i: (i, 0)),
            pl.BlockSpec((8, H), lambda i: (0, 0)),
        ],
        out_shape=[
            jax.ShapeDtypeStruct((N, H), jnp.float32),
            jax.ShapeDtypeStruct((N, H), jnp.float32),
            jax.ShapeDtypeStruct((8, H), jnp.float32),
        ],
        scratch_shapes=[pltpu.VMEM((8, H), jnp.float32)],
    )(x, w1, b1.reshape(1, H), w2, b2.reshape(1, H))


def _pack_half(yh):
    a = lax.bitcast_convert_type(yh[:, :HQ].astype(jnp.bfloat16),
                                 jnp.uint16).astype(jnp.uint32)
    b = lax.bitcast_convert_type(yh[:, HQ:].astype(jnp.bfloat16),
                                 jnp.uint16).astype(jnp.uint32)
    return a | (b << 16)


def _finish_split_body(h1_ref, h2_ref, st_ref,
                       g1_ref, be1_ref, w1b_ref, b1b_ref,
                       g2_ref, be2_ref, w2b_ref, b2b_ref,
                       y1lo_ref, y1hi_ref, y2lo_ref, y2hi_ref):
    st = st_ref[...]
    inv_n = jnp.float32(1.0 / N)

    def norm_relu(h, srow, g, be):
        mu = st[srow:srow + 1] * inv_n
        var = st[srow + 1:srow + 2] * inv_n - mu * mu
        return jax.nn.relu((h - mu) / jnp.sqrt(var + 1e-5) * g + be)

    a1 = norm_relu(h1_ref[...], 0, g1_ref[...], be1_ref[...])
    y1 = jnp.dot(a1, w1b_ref[...], preferred_element_type=jnp.float32) + b1b_ref[...]
    y1lo_ref[...] = _pack_half(y1[:, :HH])
    y1hi_ref[...] = _pack_half(y1[:, HH:])
    a2 = norm_relu(h2_ref[...], 2, g2_ref[...], be2_ref[...])
    y2 = jnp.dot(a2, w2b_ref[...], preferred_element_type=jnp.float32) + b2b_ref[...]
    y2lo_ref[...] = _pack_half(y2[:, :HH])
    y2hi_ref[...] = _pack_half(y2[:, HH:])


def _finish_split(h1, h2, st, g1, be1, w1b, b1b, g2, be2, w2b, b2b):
    grid = (NB,)
    vec = pl.BlockSpec((1, H), lambda i: (0, 0))
    mat = pl.BlockSpec((H, H), lambda i: (0, 0))
    blk = pl.BlockSpec((RB, H), lambda i: (i, 0))
    half = pl.BlockSpec((RB, HQ), lambda i: (i, 0))
    return pl.pallas_call(
        _finish_split_body,
        grid=grid,
        in_specs=[blk, blk, pl.BlockSpec((8, H), lambda i: (0, 0)),
                  vec, vec, mat, vec, vec, vec, mat, vec],
        out_specs=[half, half, half, half],
        out_shape=[jax.ShapeDtypeStruct((N, HQ), jnp.uint32)] * 4,
    )(h1, h2, st, g1.reshape(1, H), be1.reshape(1, H), w1b,
      b1b.reshape(1, H), g2.reshape(1, H), be2.reshape(1, H), w2b,
      b2b.reshape(1, H))


def _sc_body(y1lo, y1hi, y2lo, y2hi, i0r, i1r, i2r, out_hbm,
             ib0, ib1, ib2, r1, r2, prod, accum, gsa, gsb, ssa, ssb):
    c = lax.axis_index("c")
    s = lax.axis_index("s")

    zero = jnp.zeros((16,), jnp.float32)

    def zrow(r, _):
        for j in range(HH // 16):
            prod[0, r, pl.ds(j * 16, 16)] = zero
        return 0

    lax.fori_loop(0, CH, zrow, 0)
    for k in range(RPT // CH):
        pltpu.sync_copy(prod.at[0], accum.at[pl.ds(s * RPT + k * CH, CH)])
    plsc.subcore_barrier()

    gsems = (gsa, gsb)
    ssems = (ssa, ssb)

    def process(t1, t2):
        def fire(ch, b):
            pltpu.async_copy(t1.at[ib1.at[pl.ds(ch * CH, CH)]], r1.at[b],
                             gsems[b])
            pltpu.async_copy(t2.at[ib2.at[pl.ds(ch * CH, CH)]], r2.at[b],
                             gsems[b])

        def wait_scatter(b):
            pltpu.make_async_copy(prod.at[b], accum.at[ib0.at[0]],
                                  ssems[b]).wait()

        def drain(ch, b):
            pltpu.make_async_copy(t1.at[ib1.at[pl.ds(ch * CH, CH)]],
                                  r1.at[b], gsems[b]).wait()
            pltpu.make_async_copy(t2.at[ib2.at[pl.ds(ch * CH, CH)]],
                                  r2.at[b], gsems[b]).wait()

            @pl.when(ch >= 2)
            def _():
                wait_scatter(b)

            @plsc.parallel_loop(0, CH, 1, unroll=4)
            def _(r):
                m = jnp.uint32(0xFFFF0000)
                for j in range(HQ // 16):
                    sl = pl.ds(j * 16, 16)
                    v1 = r1[b, r, sl]
                    v2 = r2[b, r, sl]
                    a1 = lax.bitcast_convert_type(v1 << 16, jnp.float32)
                    a2 = lax.bitcast_convert_type(v2 << 16, jnp.float32)
                    b1 = lax.bitcast_convert_type(v1 & m, jnp.float32)
                    b2 = lax.bitcast_convert_type(v2 & m, jnp.float32)
                    prod[b, r, pl.ds(j * 16, 16)] = a1 * a2
                    prod[b, r, pl.ds(HQ + j * 16, 16)] = b1 * b2

            pltpu.async_copy(prod.at[b], accum.at[ib0.at[ch]], ssems[b],
                             add=True)

        def super_blk(sb, _):
            pltpu.sync_copy(i0r.at[s * NSB + sb], ib0)
            base = s * EPT + sb * (SB * CH)
            pltpu.sync_copy(i1r.at[pl.ds(base, SB * CH)], ib1)
            pltpu.sync_copy(i2r.at[pl.ds(base, SB * CH)], ib2)

            fire(0, 0)
            fire(1, 1)

            def pair(g, _):
                ch = 2 * g
                drain(ch, 0)

                @pl.when(ch + 2 < SB)
                def _():
                    fire(ch + 2, 0)

                drain(ch + 1, 1)

                @pl.when(ch + 3 < SB)
                def _():
                    fire(ch + 3, 1)

                return 0

            lax.fori_loop(0, SB // 2, pair, 0)
            if SB % 2:
                drain(SB - 1, 0)
            wait_scatter(0)
            wait_scatter(1)
            return 0

        lax.fori_loop(0, NSB, super_blk, 0)

    @pl.when(c == 0)
    def _():
        process(y1lo, y2lo)

    @pl.when(c == 1)
    def _():
        process(y1hi, y2hi)

    plsc.subcore_barrier()
    for k in range(RPT // CH):
        row0 = s * RPT + k * CH
        pltpu.sync_copy(accum.at[pl.ds(row0, CH)],
                        out_hbm.at[c].at[pl.ds(row0, CH)])


def _sc_edge_aggregate(y1lo, y1hi, y2lo, y2hi, i0, i1, i2):
    mesh = plsc.VectorSubcoreMesh(core_axis_name="c", subcore_axis_name="s",
                                  num_cores=NC, num_subcores=NS)
    fn = pl.kernel(
        _sc_body,
        out_type=jax.ShapeDtypeStruct((NC, NP, HH), jnp.float32),
        mesh=mesh,
        compiler_params=pltpu.CompilerParams(use_tc_tiling_on_sc=False),
        scratch_types=[
            pltpu.VMEM((SB, CH), jnp.int32),
            pltpu.VMEM((SB * CH,), jnp.int32),
            pltpu.VMEM((SB * CH,), jnp.int32),
            pltpu.VMEM((2, CH, HQ), jnp.uint32),
            pltpu.VMEM((2, CH, HQ), jnp.uint32),
            pltpu.VMEM((2, CH, HH), jnp.float32),
            pltpu.VMEM_SHARED((NP, HH), jnp.float32),
            pltpu.SemaphoreType.DMA,
            pltpu.SemaphoreType.DMA,
            pltpu.SemaphoreType.DMA,
            pltpu.SemaphoreType.DMA,
        ],
    )
    return fn(y1lo, y1hi, y2lo, y2hi,
              i0.reshape(NS * NSB, SB, CH), i1, i2)


def _upd_stats_body(nsteps, x_ref, p0_ref, p1_ref, wx_ref, w0_ref, w1_ref,
                    b_ref, hu_ref, st_ref, acc_ref):
    i = pl.program_id(0)
    hu = (jnp.dot(x_ref[...], wx_ref[...], preferred_element_type=jnp.float32)
          + jnp.dot(p0_ref[...], w0_ref[...], preferred_element_type=jnp.float32)
          + jnp.dot(p1_ref[...], w1_ref[...], preferred_element_type=jnp.float32)
          + b_ref[...])
    hu_ref[...] = hu
    part = jnp.concatenate([
        jnp.sum(hu, axis=0, keepdims=True),
        jnp.sum(hu * hu, axis=0, keepdims=True),
        jnp.zeros((6, H), jnp.float32),
    ], axis=0)

    @pl.when(i == 0)
    def _():
        acc_ref[...] = part

    @pl.when(i > 0)
    def _():
        acc_ref[...] = acc_ref[...] + part

    @pl.when(i == nsteps - 1)
    def _():
        st_ref[...] = acc_ref[...]


def _upd_stats(x, p0, p1, wx, w0, w1, b):
    grid = (NB,)
    blk = pl.BlockSpec((RB, H), lambda i: (i, 0))
    half = pl.BlockSpec((RB, HH), lambda i: (i, 0))
    return pl.pallas_call(
        functools.partial(_upd_stats_body, NB),
        grid=grid,
        in_specs=[blk, half, half,
                  pl.BlockSpec((H, H), lambda i: (0, 0)),
                  pl.BlockSpec((HH, H), lambda i: (0, 0)),
                  pl.BlockSpec((HH, H), lambda i: (0, 0)),
                  pl.BlockSpec((1, H), lambda i: (0, 0))],
        out_specs=[blk, pl.BlockSpec((8, H), lambda i: (0, 0))],
        out_shape=[
            jax.ShapeDtypeStruct((N, H), jnp.float32),
            jax.ShapeDtypeStruct((8, H), jnp.float32),
        ],
        scratch_shapes=[pltpu.VMEM((8, H), jnp.float32)],
    )(x, p0, p1, wx, w0, w1, b.reshape(1, H))


def _upd_finish_body(hu_ref, st_ref, g_ref, be_ref, w2_ref, b2_ref, x_ref,
                     out_ref):
    st = st_ref[...]
    inv_n = jnp.float32(1.0 / N)
    mu = st[0:1] * inv_n
    var = st[1:2] * inv_n - mu * mu
    a = jax.nn.relu((hu_ref[...] - mu) / jnp.sqrt(var + 1e-5) * g_ref[...]
                    + be_ref[...])
    out_ref[...] = (jnp.dot(a, w2_ref[...], preferred_element_type=jnp.float32)
                    + b2_ref[...] + x_ref[...])


def _upd_finish(hu, st, g, be, w2, b2, x):
    grid = (NB,)
    blk = pl.BlockSpec((RB, H), lambda i: (i, 0))
    vec = pl.BlockSpec((1, H), lambda i: (0, 0))
    return pl.pallas_call(
        _upd_finish_body,
        grid=grid,
        in_specs=[blk, pl.BlockSpec((8, H), lambda i: (0, 0)),
                  vec, vec, pl.BlockSpec((H, H), lambda i: (0, 0)), vec, blk],
        out_specs=blk,
        out_shape=jax.ShapeDtypeStruct((N, H), jnp.float32),
    )(hu, st, g.reshape(1, H), be.reshape(1, H), w2, b2.reshape(1, H), x)


def kernel(pair_h, tuple_index, W1a, b1a, g1, be1, W1b, b1b,
           W2a, b2a, g2, be2, W2b, b2b, Wu1, bu1, gu, beu, Wu2, bu2):
    h1, h2, st12 = _mm_stats(pair_h, W1a, b1a, W2a, b2a)
    y1lo, y1hi, y2lo, y2hi = _finish_split(
        h1, h2, st12, g1, be1, W1b, b1b, g2, be2, W2b, b2b)

    i0 = tuple_index[0]
    i1 = tuple_index[1]
    i2 = tuple_index[2]
    p = _sc_edge_aggregate(y1lo, y1hi, y2lo, y2hi, i0, i1, i2)

    hu, stu = _upd_stats(pair_h, p[0], p[1], Wu1[:H], Wu1[H:H + HH],
                         Wu1[H + HH:], bu1)
    return _upd_finish(hu, stu, gu, beu, Wu2, bu2, pair_h)
